# Initial kernel scaffold; baseline (speedup 1.0000x reference)
import jax
import jax.numpy as jnp
from jax.experimental import pallas as pl


def kernel(x_mesh, x_grid, edge_index, edge_attr, ee_w1, ee_b1, ee_w2, ee_b2, ee_g, ee_bt, em_w1, em_b1, em_w2, em_b2, em_g, em_bt, nm_w1, nm_b1, nm_w2, nm_b2, nm_g, nm_bt):
    raise NotImplementedError("write your pallas kernel here")


# speedup vs baseline: 2.0319x; 2.0319x over previous
import functools

import jax
import jax.numpy as jnp
from jax import lax
from jax.experimental import pallas as pl
from jax.experimental.pallas import tpu as pltpu
from jax.experimental.pallas import tpu_sc as plsc

N_MESH = 10000
N_GRID = 50000
N_EDGES = 800000
LATENT = 64

NC = 2
NS = 16
NW = NC * NS
CH = 128
E_PER_W = 25088
E_PAD = E_PER_W * NW
G_CHUNKS = E_PER_W // CH

SEG_PER_SC = N_GRID // NC
TRASH = SEG_PER_SC
SPM_ROWS = 25024
ZROWS_PER_TILE = SPM_ROWS // NS
E_PER_TILE = N_EDGES // NS
FULL_CH = E_PER_TILE // CH
REM = E_PER_TILE - FULL_CH * CH


def _make_gather():
    mesh = plsc.VectorSubcoreMesh(core_axis_name="c", subcore_axis_name="s")

    @functools.partial(
        pl.kernel,
        mesh=mesh,
        compiler_params=pltpu.CompilerParams(use_tc_tiling_on_sc=False),
        out_type=(
            jax.ShapeDtypeStruct((E_PAD, LATENT), jnp.float32),
            jax.ShapeDtypeStruct((E_PAD, LATENT), jnp.float32),
        ),
        scratch_types=[
            pltpu.VMEM((CH,), jnp.int32),
            pltpu.VMEM((CH,), jnp.int32),
            pltpu.VMEM((CH, LATENT), jnp.float32),
            pltpu.VMEM((CH, LATENT), jnp.float32),
            pltpu.SemaphoreType.DMA,
            pltpu.SemaphoreType.DMA,
        ],
    )
    def gather_k(xm, xg, src, dst, snd_out, rcv_out,
                 sidx, didx, srow, drow, s_sem, d_sem):
        wid = lax.axis_index("s") * NC + lax.axis_index("c")
        base = wid * E_PER_W

        def body(j, carry):
            off = base + j * CH
            pltpu.sync_copy(src.at[pl.ds(off, CH)], sidx)
            pltpu.sync_copy(dst.at[pl.ds(off, CH)], didx)
            cs = pltpu.async_copy(xm.at[sidx], srow, s_sem)
            cd = pltpu.async_copy(xg.at[didx], drow, d_sem)
            cs.wait()
            cd.wait()
            pltpu.sync_copy(srow, snd_out.at[pl.ds(off, CH)])
            pltpu.sync_copy(drow, rcv_out.at[pl.ds(off, CH)])
            return carry

        lax.fori_loop(0, G_CHUNKS, body, 0)

    return gather_k


_gather = _make_gather()


def _make_scatter():
    mesh = plsc.VectorSubcoreMesh(core_axis_name="c", subcore_axis_name="s")

    @functools.partial(
        pl.kernel,
        mesh=mesh,
        compiler_params=pltpu.CompilerParams(use_tc_tiling_on_sc=False),
        out_type=jax.ShapeDtypeStruct((N_GRID, LATENT), jnp.float32),
        scratch_types=[
            pltpu.VMEM((CH,), jnp.int32),
            pltpu.VMEM((CH, LATENT), jnp.float32),
            pltpu.VMEM((REM,), jnp.int32),
            pltpu.VMEM((REM, LATENT), jnp.float32),
            pltpu.VMEM_SHARED((SPM_ROWS, LATENT), jnp.float32),
        ],
    )
    def scatter_k(ue, dst, zeros, agg_out, idx, rows, idx_r, rows_r, acc):
        cid = lax.axis_index("c")
        tid = lax.axis_index("s")
        seg0 = cid * SEG_PER_SC

        zrow0 = tid * ZROWS_PER_TILE
        for z in range(ZROWS_PER_TILE // CH):
            pltpu.sync_copy(zeros, acc.at[pl.ds(zrow0 + z * CH, CH)])
        ztail = ZROWS_PER_TILE - (ZROWS_PER_TILE // CH) * CH
        if ztail:
            pltpu.sync_copy(zeros.at[pl.ds(0, ztail)],
                            acc.at[pl.ds(zrow0 + (ZROWS_PER_TILE // CH) * CH, ztail)])
        plsc.subcore_barrier()

        ebase = tid * E_PER_TILE

        def do_chunk(off, n, idx_b, rows_b):
            pltpu.sync_copy(dst.at[pl.ds(off, n)], idx_b)
            pltpu.sync_copy(ue.at[pl.ds(off, n)], rows_b)
            for g in range(n // 16):
                v = idx_b[pl.ds(g * 16, 16)]
                loc = v - seg0
                ok = (loc >= 0) & (loc < SEG_PER_SC)
                idx_b[pl.ds(g * 16, 16)] = jnp.where(ok, loc, TRASH)
            pltpu.sync_copy(rows_b, acc.at[idx_b], add=True)

        def body(j, carry):
            do_chunk(ebase + j * CH, CH, idx, rows)
            return carry

        lax.fori_loop(0, FULL_CH, body, 0)
        do_chunk(ebase + FULL_CH * CH, REM, idx_r, rows_r)
        plsc.subcore_barrier()

        @pl.when(tid < NS - 1)
        def _():
            r0 = tid * 1568
            for z in range(12):
                pltpu.sync_copy(acc.at[pl.ds(r0 + z * CH, CH)],
                                agg_out.at[pl.ds(seg0 + r0 + z * CH, CH)])
            pltpu.sync_copy(acc.at[pl.ds(r0 + 12 * CH, 32)],
                            agg_out.at[pl.ds(seg0 + r0 + 12 * CH, 32)])

        @pl.when(tid == NS - 1)
        def _():
            r0 = (NS - 1) * 1568
            for z in range(11):
                pltpu.sync_copy(acc.at[pl.ds(r0 + z * CH, CH)],
                                agg_out.at[pl.ds(seg0 + r0 + z * CH, CH)])
            pltpu.sync_copy(acc.at[pl.ds(r0 + 11 * CH, 72)],
                            agg_out.at[pl.ds(seg0 + r0 + 11 * CH, 72)])

    return scatter_k


_scatter = _make_scatter()


def _sig(x):
    return 1.0 / (1.0 + jnp.exp(-x))


def _ln(o, g, bt):
    mu = jnp.mean(o, axis=-1, keepdims=True)
    var = jnp.mean((o - mu) * (o - mu), axis=-1, keepdims=True)
    return (o - mu) * lax.rsqrt(var + 1e-5) * g + bt


def _dot(a, b):
    return jnp.dot(a, b, preferred_element_type=jnp.float32)


def _edge_body(attr_ref, s_ref, r_ref,
               eew1, eeb1, eew2, eeb2, eeg, eebt,
               emw1, emb1, emw2, emb2, emg, embt,
               ue_ref, ne_ref):
    a = attr_ref[...]
    h = _dot(a, eew1[...]) + eeb1[...]
    h = h * _sig(h)
    o = _dot(h, eew2[...]) + eeb2[...]
    ea = _ln(o, eeg[...], eebt[...])

    w1 = emw1[...]
    pre = (_dot(s_ref[...], w1[0:LATENT])
           + _dot(r_ref[...], w1[LATENT:2 * LATENT])
           + _dot(ea, w1[2 * LATENT:3 * LATENT])
           + emb1[...])
    h2 = pre * _sig(pre)
    o2 = _dot(h2, emw2[...]) + emb2[...]
    ue = _ln(o2, emg[...], embt[...])
    ue_ref[...] = ue
    ne_ref[...] = ea + ue


def _node_body(xg_ref, agg_ref, nmw1, nmb1, nmw2, nmb2, nmg, nmbt, out_ref):
    x = xg_ref[...]
    w1 = nmw1[...]
    pre = _dot(x, w1[0:LATENT]) + _dot(agg_ref[...], w1[LATENT:2 * LATENT]) + nmb1[...]
    h = pre * _sig(pre)
    o = _dot(h, nmw2[...]) + nmb2[...]
    out_ref[...] = x + _ln(o, nmg[...], nmbt[...])


E_BLK = 4000
N_BLK = 5000


def _edge_call(edge_attr, sender, receiver, *w):
    grid = (N_EDGES // E_BLK,)
    row_spec = lambda blk: pl.BlockSpec((blk, LATENT), lambda i: (i, 0))
    w_specs = [pl.BlockSpec(x.shape, lambda i: (0,) * x.ndim) for x in w]
    return pl.pallas_call(
        _edge_body,
        grid=grid,
        in_specs=[pl.BlockSpec((E_BLK, 4), lambda i: (i, 0)),
                  row_spec(E_BLK), row_spec(E_BLK)] + w_specs,
        out_specs=[row_spec(E_BLK), row_spec(E_BLK)],
        out_shape=[jax.ShapeDtypeStruct((N_EDGES, LATENT), jnp.float32),
                   jax.ShapeDtypeStruct((N_EDGES, LATENT), jnp.float32)],
    )(edge_attr, sender, receiver, *w)


def _node_call(x_grid, agg, *w):
    grid = (N_GRID // N_BLK,)
    row_spec = pl.BlockSpec((N_BLK, LATENT), lambda i: (i, 0))
    w_specs = [pl.BlockSpec(x.shape, lambda i: (0,) * x.ndim) for x in w]
    return pl.pallas_call(
        _node_body,
        grid=grid,
        in_specs=[row_spec, row_spec] + w_specs,
        out_specs=row_spec,
        out_shape=jax.ShapeDtypeStruct((N_GRID, LATENT), jnp.float32),
    )(x_grid, agg, *w)


def kernel(x_mesh, x_grid, edge_index, edge_attr,
           ee_w1, ee_b1, ee_w2, ee_b2, ee_g, ee_bt,
           em_w1, em_b1, em_w2, em_b2, em_g, em_bt,
           nm_w1, nm_b1, nm_w2, nm_b2, nm_g, nm_bt):
    src = edge_index[0]
    dst = edge_index[1]
    pad = E_PAD - N_EDGES
    srcp = jnp.concatenate([src, jnp.zeros((pad,), jnp.int32)])
    dstp = jnp.concatenate([dst, jnp.zeros((pad,), jnp.int32)])

    sender, receiver = _gather(x_mesh, x_grid, srcp, dstp)

    r2 = lambda v: v.reshape(1, -1)
    ue, new_edge = _edge_call(
        edge_attr, sender, receiver,
        ee_w1, r2(ee_b1), ee_w2, r2(ee_b2), r2(ee_g), r2(ee_bt),
        em_w1, r2(em_b1), em_w2, r2(em_b2), r2(em_g), r2(em_bt))

    zeros = jnp.zeros((CH, LATENT), jnp.float32)
    agg = _scatter(ue, dst, zeros)

    new_grid = _node_call(x_grid, agg,
                          nm_w1, r2(nm_b1), nm_w2, r2(nm_b2), r2(nm_g), r2(nm_bt))
    return (new_grid, new_edge)


# speedup vs baseline: 2.6882x; 1.3230x over previous
import functools

import jax
import jax.numpy as jnp
from jax import lax
from jax.experimental import pallas as pl
from jax.experimental.pallas import tpu as pltpu
from jax.experimental.pallas import tpu_sc as plsc

N_MESH = 10000
N_GRID = 50000
N_EDGES = 800000
LATENT = 64

NC = 2
NS = 16
NW = NC * NS
CH = 128
E_PER_W = 25088
E_PAD = E_PER_W * NW
G_CHUNKS = E_PER_W // CH

SEG_PER_SC = N_GRID // NC
TRASH = SEG_PER_SC
SPM_ROWS = 25024
ZROWS_PER_TILE = SPM_ROWS // NS
E_PER_TILE = N_EDGES // NS
FULL_CH = E_PER_TILE // CH
REM = E_PER_TILE - FULL_CH * CH

_SC_PARAMS = dict(compiler_params=pltpu.CompilerParams(use_tc_tiling_on_sc=False))


def _make_gather():
    mesh = plsc.VectorSubcoreMesh(core_axis_name="c", subcore_axis_name="s")

    @functools.partial(
        pl.kernel,
        mesh=mesh,
        out_type=jax.ShapeDtypeStruct((E_PAD, LATENT), jnp.float32),
        scratch_types=[
            pltpu.VMEM((CH,), jnp.int32), pltpu.VMEM((CH,), jnp.int32),
            pltpu.VMEM((CH,), jnp.int32), pltpu.VMEM((CH,), jnp.int32),
            pltpu.VMEM((CH, LATENT), jnp.float32),
            pltpu.VMEM((CH, LATENT), jnp.float32),
            pltpu.VMEM((CH, LATENT), jnp.float32),
            pltpu.VMEM((CH, LATENT), jnp.float32),
            pltpu.SemaphoreType.DMA, pltpu.SemaphoreType.DMA,
            pltpu.SemaphoreType.DMA, pltpu.SemaphoreType.DMA,
        ],
        **_SC_PARAMS,
    )
    def gather_k(xm, xg, src, dst, out,
                 si0, di0, si1, di1, ma0, ga0, ma1, ga1,
                 sem_m0, sem_g0, sem_m1, sem_g1):
        wid = lax.axis_index("s") * NC + lax.axis_index("c")
        base = wid * E_PER_W

        sets = ((si0, di0, ma0, ga0, sem_m0, sem_g0),
                (si1, di1, ma1, ga1, sem_m1, sem_g1))

        def load_start(j, s):
            si, di, ma, ga, sm, sg = s
            off = base + j * CH
            pltpu.sync_copy(src.at[pl.ds(off, CH)], si)
            pltpu.sync_copy(dst.at[pl.ds(off, CH)], di)
            pltpu.make_async_copy(xm.at[si], ma, sm).start()
            pltpu.make_async_copy(xg.at[di], ga, sg).start()

        def finish(j, s):
            si, di, ma, ga, sm, sg = s
            pltpu.make_async_copy(xm.at[si], ma, sm).wait()
            pltpu.make_async_copy(xg.at[di], ga, sg).wait()

            def add_row(r, carry):
                for g in range(LATENT // 16):
                    sl = pl.ds(g * 16, 16)
                    ma[r, sl] = ma[r, sl] + ga[r, sl]
                return carry

            lax.fori_loop(0, CH, add_row, 0)
            pltpu.sync_copy(ma, out.at[pl.ds(base + j * CH, CH)])

        load_start(0, sets[0])

        def body(jj, carry):
            j0 = 2 * jj
            load_start(j0 + 1, sets[1])
            finish(j0, sets[0])

            @pl.when(jj < G_CHUNKS // 2 - 1)
            def _():
                load_start(j0 + 2, sets[0])

            finish(j0 + 1, sets[1])
            return carry

        lax.fori_loop(0, G_CHUNKS // 2, body, 0)

    return gather_k


_gather = _make_gather()


def _make_scatter():
    mesh = plsc.VectorSubcoreMesh(core_axis_name="c", subcore_axis_name="s")

    @functools.partial(
        pl.kernel,
        mesh=mesh,
        out_type=jax.ShapeDtypeStruct((N_GRID, LATENT), jnp.float32),
        scratch_types=[
            pltpu.VMEM((CH,), jnp.int32),
            pltpu.VMEM((CH, LATENT), jnp.float32),
            pltpu.VMEM((CH,), jnp.int32),
            pltpu.VMEM((CH, LATENT), jnp.float32),
            pltpu.VMEM((REM,), jnp.int32),
            pltpu.VMEM((REM, LATENT), jnp.float32),
            pltpu.VMEM_SHARED((SPM_ROWS, LATENT), jnp.float32),
            pltpu.SemaphoreType.DMA, pltpu.SemaphoreType.DMA,
            pltpu.SemaphoreType.DMA,
        ],
        **_SC_PARAMS,
    )
    def scatter_k(ue, dst, zeros, agg_out,
                  idx0, rows0, idx1, rows1, idx_r, rows_r, acc,
                  sem0, sem1, sem_r):
        cid = lax.axis_index("c")
        tid = lax.axis_index("s")
        seg0 = cid * SEG_PER_SC

        zrow0 = tid * ZROWS_PER_TILE
        for z in range(ZROWS_PER_TILE // CH):
            pltpu.sync_copy(zeros, acc.at[pl.ds(zrow0 + z * CH, CH)])
        ztail = ZROWS_PER_TILE - (ZROWS_PER_TILE // CH) * CH
        if ztail:
            pltpu.sync_copy(zeros.at[pl.ds(0, ztail)],
                            acc.at[pl.ds(zrow0 + (ZROWS_PER_TILE // CH) * CH, ztail)])
        plsc.subcore_barrier()

        ebase = tid * E_PER_TILE
        sets = ((idx0, rows0, sem0), (idx1, rows1, sem1))

        def load_start(j, s):
            idx_b, rows_b, sem = s
            off = ebase + j * CH
            pltpu.sync_copy(dst.at[pl.ds(off, CH)], idx_b)
            pltpu.make_async_copy(ue.at[pl.ds(off, CH)], rows_b, sem).start()

        def localize(idx_b, n):
            for g in range(n // 16):
                sl = pl.ds(g * 16, 16)
                v = idx_b[sl]
                loc = v - seg0
                ok = (loc >= 0) & (loc < SEG_PER_SC)
                idx_b[sl] = jnp.where(ok, loc, TRASH)

        def finish(j, s):
            idx_b, rows_b, sem = s
            off = ebase + j * CH
            localize(idx_b, CH)
            pltpu.make_async_copy(ue.at[pl.ds(off, CH)], rows_b, sem).wait()
            pltpu.sync_copy(rows_b, acc.at[idx_b], add=True)

        load_start(0, sets[0])

        def body(jj, carry):
            j0 = 2 * jj
            load_start(j0 + 1, sets[1])
            finish(j0, sets[0])

            @pl.when(jj < FULL_CH // 2 - 1)
            def _():
                load_start(j0 + 2, sets[0])

            finish(j0 + 1, sets[1])
            return carry

        lax.fori_loop(0, FULL_CH // 2, body, 0)

        offr = ebase + FULL_CH * CH
        pltpu.sync_copy(dst.at[pl.ds(offr, REM)], idx_r)
        pltpu.make_async_copy(ue.at[pl.ds(offr, REM)], rows_r, sem_r).start()
        localize(idx_r, REM)
        pltpu.make_async_copy(ue.at[pl.ds(offr, REM)], rows_r, sem_r).wait()
        pltpu.sync_copy(rows_r, acc.at[idx_r], add=True)
        plsc.subcore_barrier()

        @pl.when(tid < NS - 1)
        def _():
            r0 = tid * 1568
            for z in range(12):
                pltpu.sync_copy(acc.at[pl.ds(r0 + z * CH, CH)],
                                agg_out.at[pl.ds(seg0 + r0 + z * CH, CH)])
            pltpu.sync_copy(acc.at[pl.ds(r0 + 12 * CH, 32)],
                            agg_out.at[pl.ds(seg0 + r0 + 12 * CH, 32)])

        @pl.when(tid == NS - 1)
        def _():
            r0 = (NS - 1) * 1568
            for z in range(11):
                pltpu.sync_copy(acc.at[pl.ds(r0 + z * CH, CH)],
                                agg_out.at[pl.ds(seg0 + r0 + z * CH, CH)])
            pltpu.sync_copy(acc.at[pl.ds(r0 + 11 * CH, 72)],
                            agg_out.at[pl.ds(seg0 + r0 + 11 * CH, 72)])

    return scatter_k


_scatter = _make_scatter()


def _sig(x):
    return 1.0 / (1.0 + jnp.exp(-x))


def _ln(o, g, bt):
    mu = jnp.mean(o, axis=-1, keepdims=True)
    var = jnp.mean((o - mu) * (o - mu), axis=-1, keepdims=True)
    return (o - mu) * lax.rsqrt(var + 1e-5) * g + bt


def _dot(a, b):
    return jnp.dot(a, b, preferred_element_type=jnp.float32)


def _prep_body(x_ref, w_ref, b_ref, out_ref):
    out_ref[...] = _dot(x_ref[...], w_ref[...]) + b_ref[...]


def _edge_body(attr_ref, gpre_ref,
               eew1, eeb1, eew2, eeb2, eeg, eebt,
               emw1, emw2, emb2, emg, embt,
               ue_ref, ne_ref):
    a = attr_ref[...]
    h = _dot(a, eew1[...]) + eeb1[...]
    h = h * _sig(h)
    o = _dot(h, eew2[...]) + eeb2[...]
    ea = _ln(o, eeg[...], eebt[...])

    pre = gpre_ref[...] + _dot(ea, emw1[...][2 * LATENT:3 * LATENT])
    h2 = pre * _sig(pre)
    o2 = _dot(h2, emw2[...]) + emb2[...]
    ue = _ln(o2, emg[...], embt[...])
    ue_ref[...] = ue
    ne_ref[...] = ea + ue


def _node_body(xg_ref, agg_ref, nmw1, nmb1, nmw2, nmb2, nmg, nmbt, out_ref):
    x = xg_ref[...]
    w1 = nmw1[...]
    pre = _dot(x, w1[0:LATENT]) + _dot(agg_ref[...], w1[LATENT:2 * LATENT]) + nmb1[...]
    h = pre * _sig(pre)
    o = _dot(h, nmw2[...]) + nmb2[...]
    out_ref[...] = x + _ln(o, nmg[...], nmbt[...])


E_BLK = 4000
N_BLK = 5000


def _prep_call(x, w, b):
    n = x.shape[0]
    blk = min(n, N_BLK)
    grid = (n // blk,)
    w_spec = pl.BlockSpec(w.shape, lambda i: (0, 0))
    b_spec = pl.BlockSpec(b.shape, lambda i: (0, 0))
    row_spec = pl.BlockSpec((blk, LATENT), lambda i: (i, 0))
    return pl.pallas_call(
        _prep_body,
        grid=grid,
        in_specs=[row_spec, w_spec, b_spec],
        out_specs=row_spec,
        out_shape=jax.ShapeDtypeStruct((n, LATENT), jnp.float32),
    )(x, w, b)


def _edge_call(edge_attr, gpre, *w):
    grid = (N_EDGES // E_BLK,)
    row_spec = pl.BlockSpec((E_BLK, LATENT), lambda i: (i, 0))
    w_specs = [pl.BlockSpec(x.shape, lambda i: (0,) * x.ndim) for x in w]
    return pl.pallas_call(
        _edge_body,
        grid=grid,
        in_specs=[pl.BlockSpec((E_BLK, 4), lambda i: (i, 0)), row_spec] + w_specs,
        out_specs=[row_spec, row_spec],
        out_shape=[jax.ShapeDtypeStruct((N_EDGES, LATENT), jnp.float32),
                   jax.ShapeDtypeStruct((N_EDGES, LATENT), jnp.float32)],
    )(edge_attr, gpre, *w)


def _node_call(x_grid, agg, *w):
    grid = (N_GRID // N_BLK,)
    row_spec = pl.BlockSpec((N_BLK, LATENT), lambda i: (i, 0))
    w_specs = [pl.BlockSpec(x.shape, lambda i: (0,) * x.ndim) for x in w]
    return pl.pallas_call(
        _node_body,
        grid=grid,
        in_specs=[row_spec, row_spec] + w_specs,
        out_specs=row_spec,
        out_shape=jax.ShapeDtypeStruct((N_GRID, LATENT), jnp.float32),
    )(x_grid, agg, *w)


def kernel(x_mesh, x_grid, edge_index, edge_attr,
           ee_w1, ee_b1, ee_w2, ee_b2, ee_g, ee_bt,
           em_w1, em_b1, em_w2, em_b2, em_g, em_bt,
           nm_w1, nm_b1, nm_w2, nm_b2, nm_g, nm_bt):
    src = edge_index[0]
    dst = edge_index[1]
    pad = E_PAD - N_EDGES
    srcp = jnp.concatenate([src, jnp.zeros((pad,), jnp.int32)])
    dstp = jnp.concatenate([dst, jnp.zeros((pad,), jnp.int32)])

    r2 = lambda v: v.reshape(1, -1)
    zrow = jnp.zeros((1, LATENT), jnp.float32)

    xm_t = _prep_call(x_mesh, em_w1[0:LATENT], r2(em_b1))
    xg_t = _prep_call(x_grid, em_w1[LATENT:2 * LATENT], zrow)

    gpre = _gather(xm_t, xg_t, srcp, dstp)

    ue, new_edge = _edge_call(
        edge_attr, gpre,
        ee_w1, r2(ee_b1), ee_w2, r2(ee_b2), r2(ee_g), r2(ee_bt),
        em_w1, em_w2, r2(em_b2), r2(em_g), r2(em_bt))

    zeros = jnp.zeros((CH, LATENT), jnp.float32)
    agg = _scatter(ue, dst, zeros)

    new_grid = _node_call(x_grid, agg,
                          nm_w1, r2(nm_b1), nm_w2, r2(nm_b2), r2(nm_g), r2(nm_bt))
    return (new_grid, new_edge)
